# baseline (speedup 1.0000x reference)
import functools
import math

import jax
import jax.numpy as jnp
from jax import lax
from jax.experimental import pallas as pl
from jax.experimental.pallas import tpu as pltpu
from jax.experimental.pallas import tpu_sc as plsc

VOCAB = 1000000
EMBED = 64
WINDOW = 200
BATCH = 4096

NUM_CORES = 2
NUM_SUBCORES = 16
NUM_WORKERS = NUM_CORES * NUM_SUBCORES

BCHUNK = BATCH // NUM_WORKERS
SCALE = math.sqrt(EMBED)


def _sc_embed(xf, t2, pf):
    mesh = plsc.VectorSubcoreMesh(core_axis_name="c", subcore_axis_name="s")

    @functools.partial(
        pl.kernel,
        mesh=mesh,
        compiler_params=pltpu.CompilerParams(use_tc_tiling_on_sc=False,
                                             needs_layout_passes=False),
        out_type=jax.ShapeDtypeStruct(
            (WINDOW, EMBED // 8, NUM_WORKERS, 8, BCHUNK), jnp.float32),
        scratch_types=[
            pltpu.VMEM((BCHUNK,), jnp.int32),
            pltpu.VMEM((BCHUNK,), jnp.int32),
            pltpu.VMEM((BCHUNK, 128), jnp.float32),
            pltpu.VMEM((EMBED, 16), jnp.float32),
            pltpu.VMEM((EMBED // 8, 8, BCHUNK), jnp.float32),
            pltpu.VMEM((WINDOW * EMBED,), jnp.float32),
            pltpu.SemaphoreType.DMA,
        ],
    )
    def k(xf_hbm, t2_hbm, pf_hbm, out_hbm, idxs, rows2, gbuf, psl, slab,
          pos_v, sem_g):
        wid = lax.axis_index("s") * NUM_CORES + lax.axis_index("c")
        b0 = wid * BCHUNK
        iota = lax.iota(jnp.int32, 16)
        rows_c = [iota + 16 * kk for kk in range(BCHUNK // 16)]
        pltpu.sync_copy(pf_hbm, pos_v)

        def per_w(w, _):
            pltpu.sync_copy(xf_hbm.at[pl.ds(w * BATCH + b0, BCHUNK)], idxs)
            h64 = []
            for kk in range(BCHUNK // 16):
                v = idxs[pl.ds(kk * 16, 16)]
                rows2[pl.ds(kk * 16, 16)] = lax.shift_right_logical(v, 1)
                h64.append(lax.shift_left(jnp.bitwise_and(v, 1), 6))
            pltpu.async_copy(t2_hbm.at[rows2], gbuf, sem_g).wait()

            def pose(e, _):
                psl[e] = plsc.load_gather(
                    pos_v, [jnp.full((16,), w * EMBED + e, jnp.int32)])
                return 0

            lax.fori_loop(0, EMBED, pose, 0)

            def inner(e, _):
                pe = psl[e]
                for kk in range(BCHUNK // 16):
                    cols = h64[kk] + e
                    v = plsc.load_gather(gbuf, [rows_c[kk], cols])
                    slab[e >> 3, e & 7, pl.ds(kk * 16, 16)] = v * SCALE + pe
                return 0

            lax.fori_loop(0, EMBED, inner, 0)
            pltpu.sync_copy(slab, out_hbm.at[w, pl.ds(0, EMBED // 8), wid])
            return 0

        lax.fori_loop(0, WINDOW, per_w, 0)

    return k(xf, t2, pf)


def kernel(x, table, pos_enc):
    xf = jnp.transpose(x.astype(jnp.int32)).reshape(-1)
    t2 = table.reshape(VOCAB // 2, 2 * EMBED)
    pf = pos_enc.reshape(-1)
    out5 = _sc_embed(xf, t2, pf)
    return out5.transpose(2, 4, 0, 1, 3).reshape(BATCH, WINDOW, EMBED)


# speedup vs baseline: 1.4381x; 1.4381x over previous
import functools
import math

import jax
import jax.numpy as jnp
from jax import lax
from jax.experimental import pallas as pl
from jax.experimental.pallas import tpu as pltpu
from jax.experimental.pallas import tpu_sc as plsc

VOCAB = 1000000
EMBED = 64
WINDOW = 200
BATCH = 4096

NUM_CORES = 2
NUM_SUBCORES = 16
NUM_WORKERS = NUM_CORES * NUM_SUBCORES

BCHUNK = BATCH // NUM_WORKERS
PADROW = 2 * EMBED
SCALE = math.sqrt(EMBED)


def _sc_embed(x2d, t2, pf):
    mesh = plsc.VectorSubcoreMesh(core_axis_name="c", subcore_axis_name="s")

    @functools.partial(
        pl.kernel,
        mesh=mesh,
        compiler_params=pltpu.CompilerParams(use_tc_tiling_on_sc=True,
                                             needs_layout_passes=False),
        out_type=jax.ShapeDtypeStruct(
            (WINDOW, EMBED // 8, NUM_WORKERS, 8, BCHUNK), jnp.float32),
        scratch_types=[
            pltpu.VMEM((WINDOW, BCHUNK), jnp.int32),
            pltpu.VMEM((2, BCHUNK, PADROW), jnp.float32),
            pltpu.VMEM((2, EMBED // 8, 8, BCHUNK), jnp.float32),
            pltpu.VMEM((WINDOW * EMBED,), jnp.float32),
            pltpu.SemaphoreType.DMA,
            pltpu.SemaphoreType.DMA,
        ],
    )
    def k(x_hbm, t2_hbm, pf_hbm, out_hbm, idx_all, gbuf, slab, pos_v,
          sem_g, sem_s):
        wid = lax.axis_index("s") * NUM_CORES + lax.axis_index("c")
        b0 = wid * BCHUNK
        iota = lax.iota(jnp.int32, 16)
        ehi_c = [lax.shift_right_logical(q * 16 + iota, 3) for q in range(4)]
        elo_c = [jnp.bitwise_and(q * 16 + iota, 7) for q in range(4)]

        pltpu.sync_copy(pf_hbm, pos_v)
        pltpu.sync_copy(x_hbm.at[pl.ds(0, WINDOW), pl.ds(b0, BCHUNK)],
                        idx_all)

        def start_gather(w, p):
            pltpu.async_copy(t2_hbm.at[idx_all.at[w]], gbuf.at[p], sem_g)

        def wait_gather(p):
            pltpu.make_async_copy(t2_hbm.at[pl.ds(0, BCHUNK)], gbuf.at[p],
                                  sem_g).wait()

        def start_scatter(w, p):
            pltpu.async_copy(slab.at[p],
                             out_hbm.at[w, pl.ds(0, EMBED // 8), wid], sem_s)

        def wait_scatter(p):
            pltpu.make_async_copy(slab.at[p],
                                  out_hbm.at[0, pl.ds(0, EMBED // 8), 0],
                                  sem_s).wait()

        def compute(w, p):
            pv = [pos_v[pl.ds(w * EMBED + q * 16, 16)] for q in range(4)]

            def rows(i2, _):
                for dr in range(2):
                    i = i2 * 2 + dr
                    ib = jnp.full((16,), i, jnp.int32)
                    for q in range(4):
                        v = gbuf[p, i, pl.ds(q * 16, 16)]
                        plsc.store_scatter(
                            slab.at[p], [ehi_c[q], elo_c[q], ib],
                            v * SCALE + pv[q])
                return 0

            lax.fori_loop(0, BCHUNK // 2, rows, 0)

        start_gather(0, 0)
        start_gather(1, 1)
        for w in range(2):
            p = w % 2
            wait_gather(p)
            compute(w, p)
            start_gather(w + 2, p)
            start_scatter(w, p)

        def body(t, _):
            for dr in range(2):
                w = t * 2 + dr
                p = dr
                wait_gather(p)
                wait_scatter(p)
                compute(w, p)
                start_gather(w + 2, p)
                start_scatter(w, p)
            return 0

        lax.fori_loop(1, WINDOW // 2 - 1, body, 0)

        for w in range(WINDOW - 2, WINDOW):
            p = w % 2
            wait_gather(p)
            wait_scatter(p)
            compute(w, p)
            start_scatter(w, p)

        wait_scatter(0)
        wait_scatter(1)

    return k(x2d, t2, pf)


def kernel(x, table, pos_enc):
    x2d = jnp.transpose(x.astype(jnp.int32))
    t2 = jnp.pad(table, ((0, 0), (0, PADROW - EMBED)))
    pf = pos_enc.reshape(-1)
    out5 = _sc_embed(x2d, t2, pf)
    return out5.transpose(2, 4, 0, 1, 3).reshape(BATCH, WINDOW, EMBED)
